# Initial kernel scaffold; baseline (speedup 1.0000x reference)
import jax
import jax.numpy as jnp
from jax.experimental import pallas as pl


def kernel(x, edge_index, edge_attr, sender_params, receiver_params, edge_params, node_params):
    raise NotImplementedError("write your pallas kernel here")


# speedup vs baseline: 3.1819x; 3.1819x over previous
import functools

import jax
import jax.numpy as jnp
from jax import lax
from jax.experimental import pallas as pl
from jax.experimental.pallas import tpu as pltpu
from jax.experimental.pallas import tpu_sc as plsc

_NC = 2
_NS = 16
_NW = _NC * _NS
_L = 16


def _mlp_rows(x, params, block):
    M, H0 = x.shape
    n = len(params)
    grid = M // block
    assert grid * block == M

    def body(x_ref, *refs):
        out_ref = refs[-1]
        h = x_ref[...]
        for i in range(n):
            W = refs[2 * i][...]
            b = refs[2 * i + 1][...]
            h = jnp.dot(h, W, preferred_element_type=jnp.float32) + b
            if i < n - 1:
                h = jnp.maximum(h, 0.0)
        out_ref[...] = h

    in_specs = [pl.BlockSpec((block, H0), lambda i: (i, 0))]
    flat = []
    for W, b in params:
        in_specs.append(pl.BlockSpec(W.shape, lambda i: (0, 0)))
        in_specs.append(pl.BlockSpec((1, W.shape[1]), lambda i: (0, 0)))
        flat += [W, b.reshape(1, -1)]
    out_dim = params[-1][0].shape[1]
    return pl.pallas_call(
        body,
        grid=(grid,),
        in_specs=in_specs,
        out_specs=pl.BlockSpec((block, out_dim), lambda i: (i, 0)),
        out_shape=jax.ShapeDtypeStruct((M, out_dim), jnp.float32),
    )(x, *flat)


def _node_mlp(x, aggp, params, block):
    M, H = x.shape
    n = len(params)
    (W0, b0) = params[0]
    W0x, W0a = W0[:H], W0[H:]
    grid = M // block

    def body(x_ref, ap_ref, w0x, w0a, b0r, *refs):
        out_ref = refs[-1]
        xb = x_ref[...]
        agg = ap_ref[0] + ap_ref[1]
        h = jnp.dot(xb, w0x[...], preferred_element_type=jnp.float32)
        h = h + jnp.dot(agg, w0a[...], preferred_element_type=jnp.float32)
        h = jnp.maximum(h + b0r[...], 0.0)
        for i in range(1, n):
            W = refs[2 * (i - 1)][...]
            b = refs[2 * (i - 1) + 1][...]
            h = jnp.dot(h, W, preferred_element_type=jnp.float32) + b
            if i < n - 1:
                h = jnp.maximum(h, 0.0)
        out_ref[...] = h + xb

    in_specs = [
        pl.BlockSpec((block, H), lambda i: (i, 0)),
        pl.BlockSpec((2, block, H), lambda i: (0, i, 0)),
        pl.BlockSpec(W0x.shape, lambda i: (0, 0)),
        pl.BlockSpec(W0a.shape, lambda i: (0, 0)),
        pl.BlockSpec((1, W0.shape[1]), lambda i: (0, 0)),
    ]
    flat = [x, aggp, W0x, W0a, b0.reshape(1, -1)]
    for W, b in params[1:]:
        in_specs.append(pl.BlockSpec(W.shape, lambda i: (0, 0)))
        in_specs.append(pl.BlockSpec((1, W.shape[1]), lambda i: (0, 0)))
        flat += [W, b.reshape(1, -1)]
    out_dim = params[-1][0].shape[1]
    return pl.pallas_call(
        body,
        grid=(grid,),
        in_specs=in_specs,
        out_specs=pl.BlockSpec((block, out_dim), lambda i: (i, 0)),
        out_shape=jax.ShapeDtypeStruct((M, out_dim), jnp.float32),
    )(*flat)


@functools.lru_cache(maxsize=None)
def _sc_edges(N, E, H):
    EPW = E // _NW
    C = 80
    NCHUNK = EPW // C
    assert EPW * _NW == E and NCHUNK * C == EPW
    NP = ((N + 2047) // 2048) * 2048
    RPS = NP // _NS
    assert RPS % C == 0
    VPR = H // _L

    mesh = plsc.VectorSubcoreMesh(
        core_axis_name="c", subcore_axis_name="s",
        num_cores=_NC, num_subcores=_NS)

    @functools.partial(
        pl.kernel,
        out_type=[
            jax.ShapeDtypeStruct((E, H), jnp.float32),
            jax.ShapeDtypeStruct((_NC, NP, H), jnp.float32),
        ],
        mesh=mesh,
        scratch_types=[
            pltpu.VMEM((C,), jnp.int32),
            pltpu.VMEM((C,), jnp.int32),
            pltpu.VMEM((C, H), jnp.float32),
            pltpu.VMEM((C, H), jnp.float32),
            pltpu.VMEM((C, H), jnp.float32),
            pltpu.VMEM((C, H), jnp.float32),
            pltpu.VMEM_SHARED((NP, H), jnp.float32),
            pltpu.SemaphoreType.DMA,
            pltpu.SemaphoreType.DMA,
        ],
    )
    def k(s_hbm, r_hbm, eh_hbm, ea_hbm, src_hbm, dst_hbm,
          eout_hbm, aggp_hbm,
          src_v, dst_v, sg_v, rg_v, eh_v, ea_v, agg_sh,
          sem1, sem2):
        cid = lax.axis_index("c")
        sid = lax.axis_index("s")
        wid = sid * _NC + cid

        zero = jnp.zeros((_L,), jnp.float32)

        def zrow(r, _):
            for c8 in range(VPR):
                eh_v[r, pl.ds(c8 * _L, _L)] = zero
            return 0
        lax.fori_loop(0, C, zrow, 0)

        def zcopy(kk, _):
            pltpu.sync_copy(eh_v, agg_sh.at[pl.ds(sid * RPS + kk * C, C)])
            return 0
        lax.fori_loop(0, RPS // C, zcopy, 0)
        plsc.subcore_barrier()

        def chunk(ci, _):
            base = wid * EPW + ci * C
            pltpu.sync_copy(src_hbm.at[pl.ds(base, C)], src_v)
            pltpu.sync_copy(dst_hbm.at[pl.ds(base, C)], dst_v)
            cp1 = pltpu.async_copy(s_hbm.at[src_v], sg_v, sem1)
            cp2 = pltpu.async_copy(r_hbm.at[dst_v], rg_v, sem2)
            pltpu.sync_copy(eh_hbm.at[pl.ds(base, C)], eh_v)
            pltpu.sync_copy(ea_hbm.at[pl.ds(base, C)], ea_v)
            cp1.wait()
            cp2.wait()

            def row(r, _):
                for c8 in range(VPR):
                    s = pl.ds(c8 * _L, _L)
                    en = eh_v[r, s] + (sg_v[r, s] + rg_v[r, s])
                    rg_v[r, s] = en
                    eh_v[r, s] = en + ea_v[r, s]
                return 0
            lax.fori_loop(0, C, row, 0)

            pltpu.sync_copy(eh_v, eout_hbm.at[pl.ds(base, C)])
            pltpu.sync_copy(rg_v, agg_sh.at[dst_v], add=True)
            return 0
        lax.fori_loop(0, NCHUNK, chunk, 0)

        plsc.subcore_barrier()
        pltpu.sync_copy(agg_sh.at[pl.ds(sid * RPS, RPS)],
                        aggp_hbm.at[cid, pl.ds(sid * RPS, RPS)])

    return k


def kernel(x, edge_index, edge_attr, sender_params, receiver_params,
           edge_params, node_params):
    N, H = x.shape
    E = edge_attr.shape[0]
    src = edge_index[0]
    dst = edge_index[1]

    S = _mlp_rows(x, sender_params, 1000)
    R = _mlp_rows(x, receiver_params, 1000)
    Eh = _mlp_rows(edge_attr, edge_params, 3200)

    e_out, aggp = _sc_edges(N, E, H)(S, R, Eh, edge_attr, src, dst)

    x_out = _node_mlp(x, aggp[:, :N], node_params, 1000)
    return (x_out, e_out)


# speedup vs baseline: 4.1294x; 1.2978x over previous
import functools

import jax
import jax.numpy as jnp
from jax import lax
from jax.experimental import pallas as pl
from jax.experimental.pallas import tpu as pltpu
from jax.experimental.pallas import tpu_sc as plsc

_NC = 2
_NS = 16
_NW = _NC * _NS
_L = 16


def _mlp_rows(x, params, block):
    M, H0 = x.shape
    n = len(params)
    grid = M // block
    assert grid * block == M

    def body(x_ref, *refs):
        out_ref = refs[-1]
        h = x_ref[...]
        for i in range(n):
            W = refs[2 * i][...]
            b = refs[2 * i + 1][...]
            h = jnp.dot(h, W, preferred_element_type=jnp.float32) + b
            if i < n - 1:
                h = jnp.maximum(h, 0.0)
        out_ref[...] = h

    in_specs = [pl.BlockSpec((block, H0), lambda i: (i, 0))]
    flat = []
    for W, b in params:
        in_specs.append(pl.BlockSpec(W.shape, lambda i: (0, 0)))
        in_specs.append(pl.BlockSpec((1, W.shape[1]), lambda i: (0, 0)))
        flat += [W, b.reshape(1, -1)]
    out_dim = params[-1][0].shape[1]
    return pl.pallas_call(
        body,
        grid=(grid,),
        in_specs=in_specs,
        out_specs=pl.BlockSpec((block, out_dim), lambda i: (i, 0)),
        out_shape=jax.ShapeDtypeStruct((M, out_dim), jnp.float32),
    )(x, *flat)


def _node_mlp(x, aggp, params, block):
    M, H = x.shape
    n = len(params)
    (W0, b0) = params[0]
    W0x, W0a = W0[:H], W0[H:]
    grid = M // block

    def body(x_ref, ap_ref, w0x, w0a, b0r, *refs):
        out_ref = refs[-1]
        xb = x_ref[...]
        agg = ap_ref[0] + ap_ref[1]
        h = jnp.dot(xb, w0x[...], preferred_element_type=jnp.float32)
        h = h + jnp.dot(agg, w0a[...], preferred_element_type=jnp.float32)
        h = jnp.maximum(h + b0r[...], 0.0)
        for i in range(1, n):
            W = refs[2 * (i - 1)][...]
            b = refs[2 * (i - 1) + 1][...]
            h = jnp.dot(h, W, preferred_element_type=jnp.float32) + b
            if i < n - 1:
                h = jnp.maximum(h, 0.0)
        out_ref[...] = h + xb

    in_specs = [
        pl.BlockSpec((block, H), lambda i: (i, 0)),
        pl.BlockSpec((2, block, H), lambda i: (0, i, 0)),
        pl.BlockSpec(W0x.shape, lambda i: (0, 0)),
        pl.BlockSpec(W0a.shape, lambda i: (0, 0)),
        pl.BlockSpec((1, W0.shape[1]), lambda i: (0, 0)),
    ]
    flat = [x, aggp, W0x, W0a, b0.reshape(1, -1)]
    for W, b in params[1:]:
        in_specs.append(pl.BlockSpec(W.shape, lambda i: (0, 0)))
        in_specs.append(pl.BlockSpec((1, W.shape[1]), lambda i: (0, 0)))
        flat += [W, b.reshape(1, -1)]
    out_dim = params[-1][0].shape[1]
    return pl.pallas_call(
        body,
        grid=(grid,),
        in_specs=in_specs,
        out_specs=pl.BlockSpec((block, out_dim), lambda i: (i, 0)),
        out_shape=jax.ShapeDtypeStruct((M, out_dim), jnp.float32),
    )(*flat)


@functools.lru_cache(maxsize=None)
def _sc_edges(N, E, H):
    EPW = E // _NW
    C = 40
    NCHUNK = EPW // C
    assert EPW * _NW == E and NCHUNK * C == EPW and NCHUNK % 2 == 0
    NP = ((N + 2047) // 2048) * 2048
    RPS = NP // _NS
    assert RPS % C == 0
    VPR = H // _L

    mesh = plsc.VectorSubcoreMesh(
        core_axis_name="c", subcore_axis_name="s",
        num_cores=_NC, num_subcores=_NS)

    @functools.partial(
        pl.kernel,
        out_type=[
            jax.ShapeDtypeStruct((E, H), jnp.float32),
            jax.ShapeDtypeStruct((_NC, NP, H), jnp.float32),
        ],
        mesh=mesh,
        scratch_types=[
            [pltpu.VMEM((C,), jnp.int32)] * 2,
            [pltpu.VMEM((C,), jnp.int32)] * 2,
            [pltpu.VMEM((C, H), jnp.float32)] * 2,
            [pltpu.VMEM((C, H), jnp.float32)] * 2,
            [pltpu.VMEM((C, H), jnp.float32)] * 2,
            [pltpu.VMEM((C, H), jnp.float32)] * 2,
            pltpu.VMEM_SHARED((NP, H), jnp.float32),
            [pltpu.SemaphoreType.DMA] * 2,
            [pltpu.SemaphoreType.DMA] * 2,
            [pltpu.SemaphoreType.DMA] * 2,
            [pltpu.SemaphoreType.DMA] * 2,
        ],
    )
    def k(s_hbm, r_hbm, eh_hbm, ea_hbm, src_hbm, dst_hbm,
          eout_hbm, aggp_hbm,
          src_v, dst_v, sg_v, rg_v, eh_v, ea_v, agg_sh,
          sems, semr, seme, sema):
        cid = lax.axis_index("c")
        sid = lax.axis_index("s")
        wid = sid * _NC + cid

        zero = jnp.zeros((_L,), jnp.float32)

        def zrow(r, _):
            for c8 in range(VPR):
                eh_v[0][r, pl.ds(c8 * _L, _L)] = zero
            return 0
        lax.fori_loop(0, C, zrow, 0)

        def zcopy(kk, _):
            pltpu.sync_copy(eh_v[0], agg_sh.at[pl.ds(sid * RPS + kk * C, C)])
            return 0
        lax.fori_loop(0, RPS // C, zcopy, 0)
        plsc.subcore_barrier()

        def issue(ci, b):
            base = wid * EPW + ci * C
            pltpu.sync_copy(src_hbm.at[pl.ds(base, C)], src_v[b])
            pltpu.sync_copy(dst_hbm.at[pl.ds(base, C)], dst_v[b])
            pltpu.async_copy(s_hbm.at[src_v[b]], sg_v[b], sems[b])
            pltpu.async_copy(r_hbm.at[dst_v[b]], rg_v[b], semr[b])
            pltpu.async_copy(eh_hbm.at[pl.ds(base, C)], eh_v[b], seme[b])
            pltpu.async_copy(ea_hbm.at[pl.ds(base, C)], ea_v[b], sema[b])

        issue(0, 0)

        def pair(g, _):
            ci0 = g * 2
            for b in range(2):
                ci = ci0 + b
                nb = 1 - b

                @pl.when(ci + 1 < NCHUNK)
                def _():
                    issue(ci + 1, nb)

                pltpu.make_async_copy(s_hbm.at[src_v[b]], sg_v[b],
                                      sems[b]).wait()
                pltpu.make_async_copy(r_hbm.at[dst_v[b]], rg_v[b],
                                      semr[b]).wait()
                base = wid * EPW + ci * C
                pltpu.make_async_copy(eh_hbm.at[pl.ds(base, C)], eh_v[b],
                                      seme[b]).wait()
                pltpu.make_async_copy(ea_hbm.at[pl.ds(base, C)], ea_v[b],
                                      sema[b]).wait()

                def row(r, _):
                    for c8 in range(VPR):
                        s = pl.ds(c8 * _L, _L)
                        en = eh_v[b][r, s] + (sg_v[b][r, s] + rg_v[b][r, s])
                        rg_v[b][r, s] = en
                        eh_v[b][r, s] = en + ea_v[b][r, s]
                    return 0
                lax.fori_loop(0, C, row, 0)

                pltpu.sync_copy(eh_v[b], eout_hbm.at[pl.ds(base, C)])
                pltpu.sync_copy(rg_v[b], agg_sh.at[dst_v[b]], add=True)
            return 0
        lax.fori_loop(0, NCHUNK // 2, pair, 0)

        plsc.subcore_barrier()
        pltpu.sync_copy(agg_sh.at[pl.ds(sid * RPS, RPS)],
                        aggp_hbm.at[cid, pl.ds(sid * RPS, RPS)])

    return k


def kernel(x, edge_index, edge_attr, sender_params, receiver_params,
           edge_params, node_params):
    N, H = x.shape
    E = edge_attr.shape[0]
    src = edge_index[0]
    dst = edge_index[1]

    S = _mlp_rows(x, sender_params, 1000)
    R = _mlp_rows(x, receiver_params, 1000)
    Eh = _mlp_rows(edge_attr, edge_params, 3200)

    e_out, aggp = _sc_edges(N, E, H)(S, R, Eh, edge_attr, src, dst)

    x_out = _node_mlp(x, aggp[:, :N], node_params, 1000)
    return (x_out, e_out)


# speedup vs baseline: 4.4053x; 1.0668x over previous
import functools

import jax
import jax.numpy as jnp
from jax import lax
from jax.experimental import pallas as pl
from jax.experimental.pallas import tpu as pltpu
from jax.experimental.pallas import tpu_sc as plsc

_NC = 2
_NS = 16
_NW = _NC * _NS
_L = 16


def _mlp_rows(x, params, block):
    M, H0 = x.shape
    n = len(params)
    grid = M // block
    assert grid * block == M

    def body(x_ref, *refs):
        out_ref = refs[-1]
        h = x_ref[...]
        for i in range(n):
            W = refs[2 * i][...]
            b = refs[2 * i + 1][...]
            h = jnp.dot(h, W, preferred_element_type=jnp.float32) + b
            if i < n - 1:
                h = jnp.maximum(h, 0.0)
        out_ref[...] = h

    in_specs = [pl.BlockSpec((block, H0), lambda i: (i, 0))]
    flat = []
    for W, b in params:
        in_specs.append(pl.BlockSpec(W.shape, lambda i: (0, 0)))
        in_specs.append(pl.BlockSpec((1, W.shape[1]), lambda i: (0, 0)))
        flat += [W, b.reshape(1, -1)]
    out_dim = params[-1][0].shape[1]
    return pl.pallas_call(
        body,
        grid=(grid,),
        in_specs=in_specs,
        out_specs=pl.BlockSpec((block, out_dim), lambda i: (i, 0)),
        out_shape=jax.ShapeDtypeStruct((M, out_dim), jnp.float32),
    )(x, *flat)


def _node_mlp(x, aggp, params, block):
    M, H = x.shape
    n = len(params)
    (W0, b0) = params[0]
    W0x, W0a = W0[:H], W0[H:]
    grid = M // block

    def body(x_ref, ap_ref, w0x, w0a, b0r, *refs):
        out_ref = refs[-1]
        xb = x_ref[...]
        agg = ap_ref[0] + ap_ref[1]
        h = jnp.dot(xb, w0x[...], preferred_element_type=jnp.float32)
        h = h + jnp.dot(agg, w0a[...], preferred_element_type=jnp.float32)
        h = jnp.maximum(h + b0r[...], 0.0)
        for i in range(1, n):
            W = refs[2 * (i - 1)][...]
            b = refs[2 * (i - 1) + 1][...]
            h = jnp.dot(h, W, preferred_element_type=jnp.float32) + b
            if i < n - 1:
                h = jnp.maximum(h, 0.0)
        out_ref[...] = h + xb

    in_specs = [
        pl.BlockSpec((block, H), lambda i: (i, 0)),
        pl.BlockSpec((2, block, H), lambda i: (0, i, 0)),
        pl.BlockSpec(W0x.shape, lambda i: (0, 0)),
        pl.BlockSpec(W0a.shape, lambda i: (0, 0)),
        pl.BlockSpec((1, W0.shape[1]), lambda i: (0, 0)),
    ]
    flat = [x, aggp, W0x, W0a, b0.reshape(1, -1)]
    for W, b in params[1:]:
        in_specs.append(pl.BlockSpec(W.shape, lambda i: (0, 0)))
        in_specs.append(pl.BlockSpec((1, W.shape[1]), lambda i: (0, 0)))
        flat += [W, b.reshape(1, -1)]
    out_dim = params[-1][0].shape[1]
    return pl.pallas_call(
        body,
        grid=(grid,),
        in_specs=in_specs,
        out_specs=pl.BlockSpec((block, out_dim), lambda i: (i, 0)),
        out_shape=jax.ShapeDtypeStruct((M, out_dim), jnp.float32),
    )(*flat)


@functools.lru_cache(maxsize=None)
def _sc_edges(N, E, H):
    EPW = E // _NW
    C = 40
    NCHUNK = EPW // C
    assert EPW * _NW == E and NCHUNK * C == EPW and NCHUNK % 2 == 0
    NP = ((N + 2047) // 2048) * 2048
    RPS = NP // _NS
    assert RPS % C == 0
    VPR = H // _L

    mesh = plsc.VectorSubcoreMesh(
        core_axis_name="c", subcore_axis_name="s",
        num_cores=_NC, num_subcores=_NS)

    @functools.partial(
        pl.kernel,
        out_type=[
            jax.ShapeDtypeStruct((E, H), jnp.float32),
            jax.ShapeDtypeStruct((_NC, NP, H), jnp.float32),
        ],
        mesh=mesh,
        scratch_types=[
            [pltpu.VMEM((C,), jnp.int32)] * 2,
            [pltpu.VMEM((C,), jnp.int32)] * 2,
            [pltpu.VMEM((C, H), jnp.float32)] * 2,
            [pltpu.VMEM((C, H), jnp.float32)] * 2,
            [pltpu.VMEM((C, H), jnp.float32)] * 2,
            [pltpu.VMEM((C, H), jnp.float32)] * 2,
            pltpu.VMEM_SHARED((NP, H), jnp.float32),
            [pltpu.SemaphoreType.DMA] * 2,
            [pltpu.SemaphoreType.DMA] * 2,
            [pltpu.SemaphoreType.DMA] * 2,
            [pltpu.SemaphoreType.DMA] * 2,
            [pltpu.SemaphoreType.DMA] * 2,
            [pltpu.SemaphoreType.DMA] * 2,
        ],
    )
    def k(s_hbm, r_hbm, eh_hbm, ea_hbm, src_hbm, dst_hbm,
          eout_hbm, aggp_hbm,
          src_v, dst_v, sg_v, rg_v, eh_v, ea_v, agg_sh,
          sems, semr, seme, sema, semw, semc):
        cid = lax.axis_index("c")
        sid = lax.axis_index("s")
        wid = sid * _NC + cid

        zero = jnp.zeros((_L,), jnp.float32)

        def zrow(r, _):
            for c8 in range(VPR):
                eh_v[0][r, pl.ds(c8 * _L, _L)] = zero
            return 0
        lax.fori_loop(0, C, zrow, 0)

        def zcopy(kk, _):
            pltpu.sync_copy(eh_v[0], agg_sh.at[pl.ds(sid * RPS + kk * C, C)])
            return 0
        lax.fori_loop(0, RPS // C, zcopy, 0)
        plsc.subcore_barrier()

        def drain_wb(b):
            pltpu.make_async_copy(eh_v[b], eout_hbm.at[pl.ds(0, C)],
                                  semw[b]).wait()
            pltpu.make_async_copy(rg_v[b], agg_sh.at[dst_v[b]],
                                  semc[b]).wait()

        def issue(ci, b):
            base = wid * EPW + ci * C
            pltpu.sync_copy(src_hbm.at[pl.ds(base, C)], src_v[b])
            pltpu.sync_copy(dst_hbm.at[pl.ds(base, C)], dst_v[b])
            pltpu.async_copy(s_hbm.at[src_v[b]], sg_v[b], sems[b])
            pltpu.async_copy(r_hbm.at[dst_v[b]], rg_v[b], semr[b])
            pltpu.async_copy(eh_hbm.at[pl.ds(base, C)], eh_v[b], seme[b])
            pltpu.async_copy(ea_hbm.at[pl.ds(base, C)], ea_v[b], sema[b])

        issue(0, 0)

        def pair(g, _):
            ci0 = g * 2
            for b in range(2):
                ci = ci0 + b
                nb = 1 - b

                @pl.when(ci + 1 < NCHUNK)
                def _():

                    @pl.when(ci + 1 >= 2)
                    def _():
                        drain_wb(nb)
                    issue(ci + 1, nb)

                pltpu.make_async_copy(s_hbm.at[src_v[b]], sg_v[b],
                                      sems[b]).wait()
                pltpu.make_async_copy(r_hbm.at[dst_v[b]], rg_v[b],
                                      semr[b]).wait()
                base = wid * EPW + ci * C
                pltpu.make_async_copy(eh_hbm.at[pl.ds(base, C)], eh_v[b],
                                      seme[b]).wait()
                pltpu.make_async_copy(ea_hbm.at[pl.ds(base, C)], ea_v[b],
                                      sema[b]).wait()

                def row(r, _):
                    for c8 in range(VPR):
                        s = pl.ds(c8 * _L, _L)
                        en = eh_v[b][r, s] + (sg_v[b][r, s] + rg_v[b][r, s])
                        rg_v[b][r, s] = en
                        eh_v[b][r, s] = en + ea_v[b][r, s]
                    return 0
                lax.fori_loop(0, C, row, 0)

                pltpu.async_copy(eh_v[b], eout_hbm.at[pl.ds(base, C)],
                                 semw[b])
                pltpu.async_copy(rg_v[b], agg_sh.at[dst_v[b]], semc[b],
                                 add=True)
            return 0
        lax.fori_loop(0, NCHUNK // 2, pair, 0)
        drain_wb(0)
        drain_wb(1)

        plsc.subcore_barrier()
        pltpu.sync_copy(agg_sh.at[pl.ds(sid * RPS, RPS)],
                        aggp_hbm.at[cid, pl.ds(sid * RPS, RPS)])

    return k


def kernel(x, edge_index, edge_attr, sender_params, receiver_params,
           edge_params, node_params):
    N, H = x.shape
    E = edge_attr.shape[0]
    src = edge_index[0]
    dst = edge_index[1]

    S = _mlp_rows(x, sender_params, 1000)
    R = _mlp_rows(x, receiver_params, 1000)
    Eh = _mlp_rows(edge_attr, edge_params, 3200)

    e_out, aggp = _sc_edges(N, E, H)(S, R, Eh, edge_attr, src, dst)

    x_out = _node_mlp(x, aggp[:, :N], node_params, 1000)
    return (x_out, e_out)


# speedup vs baseline: 4.4097x; 1.0010x over previous
import functools

import jax
import jax.numpy as jnp
from jax import lax
from jax.experimental import pallas as pl
from jax.experimental.pallas import tpu as pltpu
from jax.experimental.pallas import tpu_sc as plsc

_NC = 2
_NS = 16
_NW = _NC * _NS
_L = 16


def _mlp_rows(x, param_sets, block):
    M, H0 = x.shape
    nsets = len(param_sets)
    grid = M // block
    assert grid * block == M

    def body(x_ref, *refs):
        out_refs = refs[-nsets:]
        wrefs = refs[:-nsets]
        j = 0
        for si, params in enumerate(param_sets):
            n = len(params)
            h = x_ref[...]
            for i in range(n):
                W = wrefs[j][...]
                b = wrefs[j + 1][...]
                j += 2
                h = jnp.dot(h, W, preferred_element_type=jnp.float32) + b
                if i < n - 1:
                    h = jnp.maximum(h, 0.0)
            out_refs[si][...] = h

    in_specs = [pl.BlockSpec((block, H0), lambda i: (i, 0))]
    flat = []
    for params in param_sets:
        for W, b in params:
            in_specs.append(pl.BlockSpec(W.shape, lambda i: (0, 0)))
            in_specs.append(pl.BlockSpec((1, W.shape[1]), lambda i: (0, 0)))
            flat += [W, b.reshape(1, -1)]
    out_shapes = []
    out_specs = []
    for params in param_sets:
        out_dim = params[-1][0].shape[1]
        out_shapes.append(jax.ShapeDtypeStruct((M, out_dim), jnp.float32))
        out_specs.append(pl.BlockSpec((block, out_dim), lambda i: (i, 0)))
    res = pl.pallas_call(
        body,
        grid=(grid,),
        in_specs=in_specs,
        out_specs=out_specs,
        out_shape=out_shapes,
    )(x, *flat)
    return res


def _node_mlp(x, aggp, params, block):
    M, H = x.shape
    n = len(params)
    (W0, b0) = params[0]
    W0x, W0a = W0[:H], W0[H:]
    grid = M // block

    def body(x_ref, ap_ref, w0x, w0a, b0r, *refs):
        out_ref = refs[-1]
        xb = x_ref[...]
        agg = ap_ref[0] + ap_ref[1]
        h = jnp.dot(xb, w0x[...], preferred_element_type=jnp.float32)
        h = h + jnp.dot(agg, w0a[...], preferred_element_type=jnp.float32)
        h = jnp.maximum(h + b0r[...], 0.0)
        for i in range(1, n):
            W = refs[2 * (i - 1)][...]
            b = refs[2 * (i - 1) + 1][...]
            h = jnp.dot(h, W, preferred_element_type=jnp.float32) + b
            if i < n - 1:
                h = jnp.maximum(h, 0.0)
        out_ref[...] = h + xb

    in_specs = [
        pl.BlockSpec((block, H), lambda i: (i, 0)),
        pl.BlockSpec((2, block, H), lambda i: (0, i, 0)),
        pl.BlockSpec(W0x.shape, lambda i: (0, 0)),
        pl.BlockSpec(W0a.shape, lambda i: (0, 0)),
        pl.BlockSpec((1, W0.shape[1]), lambda i: (0, 0)),
    ]
    flat = [x, aggp, W0x, W0a, b0.reshape(1, -1)]
    for W, b in params[1:]:
        in_specs.append(pl.BlockSpec(W.shape, lambda i: (0, 0)))
        in_specs.append(pl.BlockSpec((1, W.shape[1]), lambda i: (0, 0)))
        flat += [W, b.reshape(1, -1)]
    out_dim = params[-1][0].shape[1]
    return pl.pallas_call(
        body,
        grid=(grid,),
        in_specs=in_specs,
        out_specs=pl.BlockSpec((block, out_dim), lambda i: (i, 0)),
        out_shape=jax.ShapeDtypeStruct((M, out_dim), jnp.float32),
    )(*flat)


@functools.lru_cache(maxsize=None)
def _sc_edges(N, E, H):
    EPW = E // _NW
    C = 40
    NCHUNK = EPW // C
    assert EPW * _NW == E and NCHUNK * C == EPW and NCHUNK % 2 == 0
    NP = ((N + 2047) // 2048) * 2048
    RPS = NP // _NS
    assert RPS % C == 0
    VPR = H // _L

    mesh = plsc.VectorSubcoreMesh(
        core_axis_name="c", subcore_axis_name="s",
        num_cores=_NC, num_subcores=_NS)

    @functools.partial(
        pl.kernel,
        out_type=[
            jax.ShapeDtypeStruct((E, H), jnp.float32),
            jax.ShapeDtypeStruct((_NC, NP, H), jnp.float32),
        ],
        mesh=mesh,
        scratch_types=[
            [pltpu.VMEM((C,), jnp.int32)] * 2,
            [pltpu.VMEM((C,), jnp.int32)] * 2,
            [pltpu.VMEM((C, H), jnp.float32)] * 2,
            [pltpu.VMEM((C, H), jnp.float32)] * 2,
            [pltpu.VMEM((C, H), jnp.float32)] * 2,
            [pltpu.VMEM((C, H), jnp.float32)] * 2,
            pltpu.VMEM_SHARED((NP, H), jnp.float32),
            [pltpu.SemaphoreType.DMA] * 2,
            [pltpu.SemaphoreType.DMA] * 2,
            [pltpu.SemaphoreType.DMA] * 2,
            [pltpu.SemaphoreType.DMA] * 2,
            [pltpu.SemaphoreType.DMA] * 2,
            [pltpu.SemaphoreType.DMA] * 2,
        ],
    )
    def k(s_hbm, r_hbm, eh_hbm, ea_hbm, src_hbm, dst_hbm,
          eout_hbm, aggp_hbm,
          src_v, dst_v, sg_v, rg_v, eh_v, ea_v, agg_sh,
          sems, semr, seme, sema, semw, semc):
        cid = lax.axis_index("c")
        sid = lax.axis_index("s")
        wid = sid * _NC + cid

        zero = jnp.zeros((_L,), jnp.float32)

        def zrow(r, _):
            for c8 in range(VPR):
                eh_v[0][r, pl.ds(c8 * _L, _L)] = zero
            return 0
        lax.fori_loop(0, C, zrow, 0)

        def zcopy(kk, _):
            pltpu.sync_copy(eh_v[0], agg_sh.at[pl.ds(sid * RPS + kk * C, C)])
            return 0
        lax.fori_loop(0, RPS // C, zcopy, 0)
        plsc.subcore_barrier()

        def drain_wb(b):
            pltpu.make_async_copy(eh_v[b], eout_hbm.at[pl.ds(0, C)],
                                  semw[b]).wait()
            pltpu.make_async_copy(rg_v[b], agg_sh.at[dst_v[b]],
                                  semc[b]).wait()

        def issue(ci, b):
            base = wid * EPW + ci * C
            pltpu.sync_copy(src_hbm.at[pl.ds(base, C)], src_v[b])
            pltpu.sync_copy(dst_hbm.at[pl.ds(base, C)], dst_v[b])
            pltpu.async_copy(s_hbm.at[src_v[b]], sg_v[b], sems[b])
            pltpu.async_copy(r_hbm.at[dst_v[b]], rg_v[b], semr[b])
            pltpu.async_copy(eh_hbm.at[pl.ds(base, C)], eh_v[b], seme[b])
            pltpu.async_copy(ea_hbm.at[pl.ds(base, C)], ea_v[b], sema[b])

        issue(0, 0)

        def pair(g, _):
            ci0 = g * 2
            for b in range(2):
                ci = ci0 + b
                nb = 1 - b

                @pl.when(ci + 1 < NCHUNK)
                def _():

                    @pl.when(ci + 1 >= 2)
                    def _():
                        drain_wb(nb)
                    issue(ci + 1, nb)

                pltpu.make_async_copy(s_hbm.at[src_v[b]], sg_v[b],
                                      sems[b]).wait()
                pltpu.make_async_copy(r_hbm.at[dst_v[b]], rg_v[b],
                                      semr[b]).wait()
                base = wid * EPW + ci * C
                pltpu.make_async_copy(eh_hbm.at[pl.ds(base, C)], eh_v[b],
                                      seme[b]).wait()
                pltpu.make_async_copy(ea_hbm.at[pl.ds(base, C)], ea_v[b],
                                      sema[b]).wait()

                @plsc.parallel_loop(0, C, unroll=4)
                def row(r):
                    for c8 in range(VPR):
                        s = pl.ds(c8 * _L, _L)
                        en = eh_v[b][r, s] + (sg_v[b][r, s] + rg_v[b][r, s])
                        rg_v[b][r, s] = en
                        eh_v[b][r, s] = en + ea_v[b][r, s]

                pltpu.async_copy(eh_v[b], eout_hbm.at[pl.ds(base, C)],
                                 semw[b])
                pltpu.async_copy(rg_v[b], agg_sh.at[dst_v[b]], semc[b],
                                 add=True)
            return 0
        lax.fori_loop(0, NCHUNK // 2, pair, 0)
        drain_wb(0)
        drain_wb(1)

        plsc.subcore_barrier()
        pltpu.sync_copy(agg_sh.at[pl.ds(sid * RPS, RPS)],
                        aggp_hbm.at[cid, pl.ds(sid * RPS, RPS)])

    return k


def kernel(x, edge_index, edge_attr, sender_params, receiver_params,
           edge_params, node_params):
    N, H = x.shape
    E = edge_attr.shape[0]
    src = edge_index[0]
    dst = edge_index[1]

    S, R = _mlp_rows(x, [sender_params, receiver_params], 1000)
    (Eh,) = _mlp_rows(edge_attr, [edge_params], 3200)

    e_out, aggp = _sc_edges(N, E, H)(S, R, Eh, edge_attr, src, dst)

    x_out = _node_mlp(x, aggp[:, :N], node_params, 1000)
    return (x_out, e_out)


# speedup vs baseline: 4.6155x; 1.0467x over previous
import functools

import jax
import jax.numpy as jnp
from jax import lax
from jax.experimental import pallas as pl
from jax.experimental.pallas import tpu as pltpu
from jax.experimental.pallas import tpu_sc as plsc

_NC = 2
_NS = 16
_NW = _NC * _NS
_L = 16


def _mlp_rows(x, param_sets, block):
    M, H0 = x.shape
    nsets = len(param_sets)
    grid = M // block
    assert grid * block == M

    def body(x_ref, *refs):
        out_refs = refs[-nsets:]
        wrefs = refs[:-nsets]
        j = 0
        for si, params in enumerate(param_sets):
            n = len(params)
            h = x_ref[...]
            for i in range(n):
                W = wrefs[j][...]
                b = wrefs[j + 1][...]
                j += 2
                h = jnp.dot(h, W, preferred_element_type=jnp.float32) + b
                if i < n - 1:
                    h = jnp.maximum(h, 0.0)
            out_refs[si][...] = h

    in_specs = [pl.BlockSpec((block, H0), lambda i: (i, 0))]
    flat = []
    for params in param_sets:
        for W, b in params:
            in_specs.append(pl.BlockSpec(W.shape, lambda i: (0, 0)))
            in_specs.append(pl.BlockSpec((1, W.shape[1]), lambda i: (0, 0)))
            flat += [W, b.reshape(1, -1)]
    out_shapes = []
    out_specs = []
    for params in param_sets:
        out_dim = params[-1][0].shape[1]
        out_shapes.append(jax.ShapeDtypeStruct((M, out_dim), jnp.float32))
        out_specs.append(pl.BlockSpec((block, out_dim), lambda i: (i, 0)))
    res = pl.pallas_call(
        body,
        grid=(grid,),
        in_specs=in_specs,
        out_specs=out_specs,
        out_shape=out_shapes,
    )(x, *flat)
    return res


def _edge_mlp_z(ea, params, block):
    M, H = ea.shape
    n = len(params)
    grid = M // block

    def body(x_ref, *refs):
        out_ref = refs[-1]
        xb = x_ref[...]
        h = xb
        for i in range(n):
            W = refs[2 * i][...]
            b = refs[2 * i + 1][...]
            h = jnp.dot(h, W, preferred_element_type=jnp.float32) + b
            if i < n - 1:
                h = jnp.maximum(h, 0.0)
        hb = lax.bitcast_convert_type(h.astype(jnp.bfloat16), jnp.uint16)
        xbb = lax.bitcast_convert_type(xb.astype(jnp.bfloat16), jnp.uint16)
        z = (xbb.astype(jnp.uint32) << 16) | hb.astype(jnp.uint32)
        out_ref[...] = lax.bitcast_convert_type(z, jnp.int32)

    in_specs = [pl.BlockSpec((block, H), lambda i: (i, 0))]
    flat = []
    for W, b in params:
        in_specs.append(pl.BlockSpec(W.shape, lambda i: (0, 0)))
        in_specs.append(pl.BlockSpec((1, W.shape[1]), lambda i: (0, 0)))
        flat += [W, b.reshape(1, -1)]
    return pl.pallas_call(
        body,
        grid=(grid,),
        in_specs=in_specs,
        out_specs=pl.BlockSpec((block, H), lambda i: (i, 0)),
        out_shape=jax.ShapeDtypeStruct((M, H), jnp.int32),
    )(ea, *flat)


def _node_mlp(x, aggp, params, block):
    M, H = x.shape
    n = len(params)
    (W0, b0) = params[0]
    W0x, W0a = W0[:H], W0[H:]
    grid = M // block

    def body(x_ref, ap_ref, w0x, w0a, b0r, *refs):
        out_ref = refs[-1]
        xb = x_ref[...]
        agg = ap_ref[0] + ap_ref[1]
        h = jnp.dot(xb, w0x[...], preferred_element_type=jnp.float32)
        h = h + jnp.dot(agg, w0a[...], preferred_element_type=jnp.float32)
        h = jnp.maximum(h + b0r[...], 0.0)
        for i in range(1, n):
            W = refs[2 * (i - 1)][...]
            b = refs[2 * (i - 1) + 1][...]
            h = jnp.dot(h, W, preferred_element_type=jnp.float32) + b
            if i < n - 1:
                h = jnp.maximum(h, 0.0)
        out_ref[...] = h + xb

    in_specs = [
        pl.BlockSpec((block, H), lambda i: (i, 0)),
        pl.BlockSpec((2, block, H), lambda i: (0, i, 0)),
        pl.BlockSpec(W0x.shape, lambda i: (0, 0)),
        pl.BlockSpec(W0a.shape, lambda i: (0, 0)),
        pl.BlockSpec((1, W0.shape[1]), lambda i: (0, 0)),
    ]
    flat = [x, aggp, W0x, W0a, b0.reshape(1, -1)]
    for W, b in params[1:]:
        in_specs.append(pl.BlockSpec(W.shape, lambda i: (0, 0)))
        in_specs.append(pl.BlockSpec((1, W.shape[1]), lambda i: (0, 0)))
        flat += [W, b.reshape(1, -1)]
    out_dim = params[-1][0].shape[1]
    return pl.pallas_call(
        body,
        grid=(grid,),
        in_specs=in_specs,
        out_specs=pl.BlockSpec((block, out_dim), lambda i: (i, 0)),
        out_shape=jax.ShapeDtypeStruct((M, out_dim), jnp.float32),
    )(*flat)


@functools.lru_cache(maxsize=None)
def _sc_edges(N, E, H):
    EPW = E // _NW
    C = 40
    NCHUNK = EPW // C
    assert EPW * _NW == E and NCHUNK * C == EPW and NCHUNK % 2 == 0
    NP = ((N + 2047) // 2048) * 2048
    RPS = NP // _NS
    assert RPS % C == 0
    VPR = H // _L

    mesh = plsc.VectorSubcoreMesh(
        core_axis_name="c", subcore_axis_name="s",
        num_cores=_NC, num_subcores=_NS)

    @functools.partial(
        pl.kernel,
        out_type=[
            jax.ShapeDtypeStruct((E, H), jnp.float32),
            jax.ShapeDtypeStruct((_NC, NP, H), jnp.float32),
        ],
        mesh=mesh,
        scratch_types=[
            [pltpu.VMEM((C,), jnp.int32)] * 2,
            [pltpu.VMEM((C,), jnp.int32)] * 2,
            [pltpu.VMEM((C, H), jnp.float32)] * 2,
            [pltpu.VMEM((C, H), jnp.float32)] * 2,
            [pltpu.VMEM((C, H), jnp.int32)] * 2,
            [pltpu.VMEM((C, H), jnp.float32)] * 2,
            pltpu.VMEM_SHARED((NP, H), jnp.float32),
            [pltpu.SemaphoreType.DMA] * 2,
            [pltpu.SemaphoreType.DMA] * 2,
            [pltpu.SemaphoreType.DMA] * 2,
            [pltpu.SemaphoreType.DMA] * 2,
            [pltpu.SemaphoreType.DMA] * 2,
        ],
    )
    def k(s_hbm, r_hbm, z_hbm, src_hbm, dst_hbm,
          eout_hbm, aggp_hbm,
          src_v, dst_v, sg_v, rg_v, zb_v, eo_v, agg_sh,
          sems, semr, semz, semw, semc):
        cid = lax.axis_index("c")
        sid = lax.axis_index("s")
        wid = sid * _NC + cid

        zero = jnp.zeros((_L,), jnp.float32)

        def zrow(r, _):
            for c8 in range(VPR):
                eo_v[0][r, pl.ds(c8 * _L, _L)] = zero
            return 0
        lax.fori_loop(0, C, zrow, 0)

        def zcopy(kk, _):
            pltpu.sync_copy(eo_v[0], agg_sh.at[pl.ds(sid * RPS + kk * C, C)])
            return 0
        lax.fori_loop(0, RPS // C, zcopy, 0)
        plsc.subcore_barrier()

        def drain_wb(b):
            pltpu.make_async_copy(eo_v[b], eout_hbm.at[pl.ds(0, C)],
                                  semw[b]).wait()
            pltpu.make_async_copy(rg_v[b], agg_sh.at[dst_v[b]],
                                  semc[b]).wait()

        def issue(ci, b):
            base = wid * EPW + ci * C
            pltpu.sync_copy(src_hbm.at[pl.ds(base, C)], src_v[b])
            pltpu.sync_copy(dst_hbm.at[pl.ds(base, C)], dst_v[b])
            pltpu.async_copy(s_hbm.at[src_v[b]], sg_v[b], sems[b])
            pltpu.async_copy(r_hbm.at[dst_v[b]], rg_v[b], semr[b])
            pltpu.async_copy(z_hbm.at[pl.ds(base, C)], zb_v[b], semz[b])

        issue(0, 0)

        def pair(g, _):
            ci0 = g * 2
            for b in range(2):
                ci = ci0 + b
                nb = 1 - b

                @pl.when(ci + 1 < NCHUNK)
                def _():

                    @pl.when(ci + 1 >= 2)
                    def _():
                        drain_wb(nb)
                    issue(ci + 1, nb)

                pltpu.make_async_copy(s_hbm.at[src_v[b]], sg_v[b],
                                      sems[b]).wait()
                pltpu.make_async_copy(r_hbm.at[dst_v[b]], rg_v[b],
                                      semr[b]).wait()
                base = wid * EPW + ci * C
                pltpu.make_async_copy(z_hbm.at[pl.ds(base, C)], zb_v[b],
                                      semz[b]).wait()

                @plsc.parallel_loop(0, C, unroll=4)
                def row(r):
                    for c8 in range(VPR):
                        s = pl.ds(c8 * _L, _L)
                        zv = zb_v[b][r, s]
                        ehv = lax.bitcast_convert_type(
                            lax.shift_left(zv, 16), jnp.float32)
                        eav = lax.bitcast_convert_type(
                            lax.bitwise_and(zv, jnp.int32(-65536)),
                            jnp.float32)
                        en = ehv + (sg_v[b][r, s] + rg_v[b][r, s])
                        rg_v[b][r, s] = en
                        eo_v[b][r, s] = en + eav

                pltpu.async_copy(eo_v[b], eout_hbm.at[pl.ds(base, C)],
                                 semw[b])
                pltpu.async_copy(rg_v[b], agg_sh.at[dst_v[b]], semc[b],
                                 add=True)
            return 0
        lax.fori_loop(0, NCHUNK // 2, pair, 0)
        drain_wb(0)
        drain_wb(1)

        plsc.subcore_barrier()
        pltpu.sync_copy(agg_sh.at[pl.ds(sid * RPS, RPS)],
                        aggp_hbm.at[cid, pl.ds(sid * RPS, RPS)])

    return k


def kernel(x, edge_index, edge_attr, sender_params, receiver_params,
           edge_params, node_params):
    N, H = x.shape
    E = edge_attr.shape[0]
    src = edge_index[0]
    dst = edge_index[1]

    S, R = _mlp_rows(x, [sender_params, receiver_params], 1000)
    Z = _edge_mlp_z(edge_attr, edge_params, 3200)

    e_out, aggp = _sc_edges(N, E, H)(S, R, Z, src, dst)

    x_out = _node_mlp(x, aggp[:, :N], node_params, 1000)
    return (x_out, e_out)
